# baseline (speedup 1.0000x reference)
import functools

import jax
import jax.numpy as jnp
from jax import lax
from jax.experimental import pallas as pl
from jax.experimental.pallas import tpu as pltpu
from jax.experimental.pallas import tpu_sc as plsc

_NC = 2
_NS = 16
_NW = _NC * _NS

_BI = 128
_BJ = 256
_PK = 4


def _sc_gather_g(table, idx2):
    m = idx2.shape[1]
    d = table.shape[1]
    bpw = m // _NS
    mesh = plsc.VectorSubcoreMesh(core_axis_name="c", subcore_axis_name="s")

    @functools.partial(
        pl.kernel,
        mesh=mesh,
        out_type=jax.ShapeDtypeStruct((m, _NC, d), table.dtype),
        scratch_types=[
            pltpu.VMEM((bpw,), jnp.int32),
            pltpu.VMEM((bpw, d), table.dtype),
            pltpu.SemaphoreType.DMA,
        ],
    )
    def gather_kernel(table_hbm, idx2_hbm, out_hbm, idx_v, rows_v, sem):
        half = lax.axis_index("c")
        rbase = lax.axis_index("s") * bpw

        @pl.when(half == 0)
        def _():
            pltpu.sync_copy(idx2_hbm.at[0, pl.ds(rbase, bpw)], idx_v)
            pltpu.async_copy(table_hbm.at[idx_v], rows_v, sem).wait()
            pltpu.sync_copy(rows_v, out_hbm.at[pl.ds(rbase, bpw), 0])

        @pl.when(half == 1)
        def _():
            pltpu.sync_copy(idx2_hbm.at[1, pl.ds(rbase, bpw)], idx_v)
            pltpu.async_copy(table_hbm.at[idx_v], rows_v, sem).wait()
            pltpu.sync_copy(rows_v, out_hbm.at[pl.ds(rbase, bpw), 1])

    return gather_kernel(table, idx2)


def _pair_ffnn_body(gi_ref, gjp_ref, wlw_ref, wrbd_ref, b1w_ref, woutbd_ref,
                    boutw_ref, out_ref):
    hw = wlw_ref.shape[1]
    ow = woutbd_ref.shape[1]
    bjp = gjp_ref.shape[0]
    l = jnp.dot(gi_ref[...], wlw_ref[...], preferred_element_type=jnp.float32)
    l = l + b1w_ref[...]
    r = jnp.dot(gjp_ref[...], wrbd_ref[...], preferred_element_type=jnp.float32)
    h = jnp.maximum(l[:, None, :] + r[None, :, :], 0.0)
    o = jnp.dot(h.reshape(_BI * bjp, hw), woutbd_ref[...],
                preferred_element_type=jnp.float32)
    o = o + boutw_ref[...]
    out_ref[...] = o.reshape(_BI, bjp, ow)


def kernel(subword_embeddings, begin_indexes, end_indexes, W1, b1, Wout, bout):
    m = begin_indexes.shape[0]
    d = subword_embeddings.shape[1]
    h_dim = W1.shape[1]
    out_dim = Wout.shape[1]
    pk = _PK

    idx2 = jnp.stack([begin_indexes, end_indexes]).astype(jnp.int32)
    g3 = _sc_gather_g(subword_embeddings, idx2)
    g = g3.reshape(m, 2 * d)
    gjp = g3.reshape(m // pk, pk * 2 * d)

    wl = W1[: 2 * d]
    wr = W1[2 * d:]
    wl_wide = jnp.concatenate([wl] * pk, axis=1)
    wr_bd = jax.scipy.linalg.block_diag(*([wr] * pk))
    wout_bd = jax.scipy.linalg.block_diag(*([Wout] * pk))
    b1_wide = jnp.concatenate([b1] * pk).reshape(1, pk * h_dim)
    bout_wide = jnp.concatenate([bout] * pk).reshape(1, pk * out_dim)

    grid = (m // _BI, m // _BJ)
    out = pl.pallas_call(
        _pair_ffnn_body,
        grid=grid,
        in_specs=[
            pl.BlockSpec((_BI, 2 * d), lambda i, j: (i, 0)),
            pl.BlockSpec((_BJ // pk, pk * 2 * d), lambda i, j: (j, 0)),
            pl.BlockSpec((2 * d, pk * h_dim), lambda i, j: (0, 0)),
            pl.BlockSpec((pk * 2 * d, pk * h_dim), lambda i, j: (0, 0)),
            pl.BlockSpec((1, pk * h_dim), lambda i, j: (0, 0)),
            pl.BlockSpec((pk * h_dim, pk * out_dim), lambda i, j: (0, 0)),
            pl.BlockSpec((1, pk * out_dim), lambda i, j: (0, 0)),
        ],
        out_specs=pl.BlockSpec((_BI, _BJ // pk, pk * out_dim),
                               lambda i, j: (i, j, 0)),
        out_shape=jax.ShapeDtypeStruct((m, m // pk, pk * out_dim), jnp.float32),
    )(g, gjp, wl_wide, wr_bd, b1_wide, wout_bd, bout_wide)
    return out.reshape(m, m, out_dim)


# speedup vs baseline: 2.2345x; 2.2345x over previous
import functools

import numpy as np

import jax
import jax.numpy as jnp
from jax import lax
from jax.experimental import pallas as pl
from jax.experimental.pallas import tpu as pltpu
from jax.experimental.pallas import tpu_sc as plsc

_NC = 2
_NS = 16
_NW = _NC * _NS

_BI = 128
_BJ = 512
_PK = 4
_CI = 32


def _sc_gather_g(table, idx2):
    m = idx2.shape[1]
    d = table.shape[1]
    bpw = m // _NS
    mesh = plsc.VectorSubcoreMesh(core_axis_name="c", subcore_axis_name="s")

    @functools.partial(
        pl.kernel,
        mesh=mesh,
        out_type=jax.ShapeDtypeStruct((m, 2 * d), table.dtype),
        scratch_types=[
            pltpu.VMEM((bpw,), jnp.int32),
            pltpu.VMEM((bpw, d), table.dtype),
            pltpu.SemaphoreType.DMA,
        ],
    )
    def gather_kernel(table_hbm, idx2_hbm, out_hbm, idx_v, rows_v, sem):
        half = lax.axis_index("c")
        rbase = lax.axis_index("s") * bpw

        @pl.when(half == 0)
        def _():
            pltpu.sync_copy(idx2_hbm.at[0, pl.ds(rbase, bpw)], idx_v)
            pltpu.async_copy(table_hbm.at[idx_v], rows_v, sem).wait()
            pltpu.sync_copy(rows_v, out_hbm.at[pl.ds(rbase, bpw), pl.ds(0, d)])

        @pl.when(half == 1)
        def _():
            pltpu.sync_copy(idx2_hbm.at[1, pl.ds(rbase, bpw)], idx_v)
            pltpu.async_copy(table_hbm.at[idx_v], rows_v, sem).wait()
            pltpu.sync_copy(rows_v, out_hbm.at[pl.ds(rbase, bpw), pl.ds(d, d)])

    return gather_kernel(table, idx2)


def _pair_ffnn_body(gi_ref, gjp_ref, wlw_ref, wrbd_ref, b1w_ref, woutp_ref,
                    boutp_ref, out_ref):
    hw = wlw_ref.shape[1]
    ow = woutp_ref.shape[1]
    bjp = gjp_ref.shape[0]
    l = jnp.dot(gi_ref[...], wlw_ref[...], preferred_element_type=jnp.float32)
    l = l + b1w_ref[...]
    r = jnp.dot(gjp_ref[...], wrbd_ref[...], preferred_element_type=jnp.float32)
    for c in range(_BI // _CI):
        lc = l[c * _CI:(c + 1) * _CI]
        h = jnp.maximum(lc[:, None, :] + r[None, :, :], 0.0)
        o = jnp.dot(h.reshape(_CI * bjp, hw), woutp_ref[...],
                    preferred_element_type=jnp.float32)
        o = o + boutp_ref[...]
        out_ref[pl.ds(c * _CI, _CI)] = jnp.transpose(
            o.reshape(_CI, bjp, ow), (0, 2, 1))


def kernel(subword_embeddings, begin_indexes, end_indexes, W1, b1, Wout, bout):
    m = begin_indexes.shape[0]
    d = subword_embeddings.shape[1]
    h_dim = W1.shape[1]
    out_dim = Wout.shape[1]
    pk = _PK
    grp = m // pk

    idx_n = jnp.stack([begin_indexes, end_indexes]).astype(jnp.int32)
    idx_p = idx_n.reshape(2, pk, grp).transpose(0, 2, 1).reshape(2, m)
    g = _sc_gather_g(subword_embeddings, idx_n)
    gp = _sc_gather_g(subword_embeddings, idx_p)
    gjp = gp.reshape(grp, pk * 2 * d)

    wl = W1[: 2 * d]
    wr = W1[2 * d:]
    wl_wide = jnp.concatenate([wl] * pk, axis=1)
    wr_bd = jax.scipy.linalg.block_diag(*([wr] * pk))
    wout_bd = jax.scipy.linalg.block_diag(*([Wout] * pk))
    b1_wide = jnp.concatenate([b1] * pk).reshape(1, pk * h_dim)
    bout_wide = jnp.concatenate([bout] * pk).reshape(1, pk * out_dim)
    cols = np.arange(pk * out_dim)
    kk = 8 * (cols // (8 * pk)) + cols % 8
    tt = (cols % (8 * pk)) // 8
    perm = tt * out_dim + kk
    wout_p = wout_bd[:, perm]
    bout_p = bout_wide[:, perm]

    grid = (m // _BI, m // _BJ)
    out = pl.pallas_call(
        _pair_ffnn_body,
        grid=grid,
        in_specs=[
            pl.BlockSpec((_BI, 2 * d), lambda i, j: (i, 0)),
            pl.BlockSpec((_BJ // pk, pk * 2 * d), lambda i, j: (j, 0)),
            pl.BlockSpec((2 * d, pk * h_dim), lambda i, j: (0, 0)),
            pl.BlockSpec((pk * 2 * d, pk * h_dim), lambda i, j: (0, 0)),
            pl.BlockSpec((1, pk * h_dim), lambda i, j: (0, 0)),
            pl.BlockSpec((pk * h_dim, pk * out_dim), lambda i, j: (0, 0)),
            pl.BlockSpec((1, pk * out_dim), lambda i, j: (0, 0)),
        ],
        out_specs=pl.BlockSpec((_BI, pk * out_dim, _BJ // pk),
                               lambda i, j: (i, 0, j)),
        out_shape=jax.ShapeDtypeStruct((m, pk * out_dim, grp), jnp.float32),
    )(g, gjp, wl_wide, wr_bd, b1_wide, wout_p, bout_p)
    out5 = out.reshape(m, out_dim // 8, pk, 8, grp)
    return out5.transpose(0, 2, 4, 1, 3).reshape(m, m, out_dim)
